# baseline (speedup 1.0000x reference)
import jax
import jax.numpy as jnp
from jax import lax
from jax.experimental import pallas as pl
from jax.experimental.pallas import tpu as pltpu
from jax.experimental.pallas import tpu_sc as plsc

N = 320000
D = 128
K = 1024
RB = 128
BLK = 256
NW = 32

SC_ROWS = 245760
NBLK = SC_ROWS // BLK
BASE_BLKS = NBLK // NW
EXTRA = NBLK - BASE_BLKS * NW
NRING = 3
TRIPS = BASE_BLKS // NRING

TCB = 512
TC_ROWS = N - SC_ROWS
TC_STEPS = TC_ROWS // TCB
TC_START = SC_ROWS // TCB


def _sc_body(emb, lbl, part_o, cnt_o, sq_o, acc_sp,
             buf0, buf1, buf2, la0, lb0, la1, lb1, la2, lb2, cnt, sqbuf,
             sin0, sin1, sin2, sl0, sl1, sl2, ssc0, ssc1, ssc2):
    c = lax.axis_index("c")
    s = lax.axis_index("s")
    wid = s * 2 + c

    zf16 = jnp.zeros((16,), jnp.float32)
    zi16 = jnp.zeros((16,), jnp.int32)
    iota16 = lax.broadcasted_iota(jnp.int32, (16,), 0)
    ones16 = jnp.ones((16,), jnp.int32)
    bufs = (buf0, buf1, buf2)
    las = (la0, la1, la2)
    lbs = (lb0, lb1, lb2)
    sins = (sin0, sin1, sin2)
    sls = (sl0, sl1, sl2)
    sscs = (ssc0, ssc1, ssc2)

    def zero_cnt(i, _):
        for j in range(16):
            cnt[j, pl.ds(i * 16, 16)] = zi16
        return 0
    lax.fori_loop(0, K // 16, zero_cnt, 0)
    sqbuf[pl.ds(0, 16)] = zf16

    def zero_buf(i, _):
        for v in range(8):
            buf0[i, pl.ds(v * 16, 16)] = zf16
        return 0
    lax.fori_loop(0, RB, zero_buf, 0)

    @pl.when(s == 0)
    def _():
        for i in range(K // RB):
            pltpu.sync_copy(buf0.at[pl.ds(0, RB)], acc_sp.at[pl.ds(i * RB, RB)])

    plsc.subcore_barrier()

    start = wid * BASE_BLKS

    def start_in(blkidx, p):
        row0 = blkidx * BLK
        pltpu.async_copy(emb.at[pl.ds(row0, BLK)], bufs[p], sins[p])
        pltpu.async_copy(lbl.at[pl.ds(row0, RB)], las[p], sls[p])
        pltpu.async_copy(lbl.at[pl.ds(row0 + RB, RB)], lbs[p], sls[p])

    def wait_in(p):
        pltpu.make_async_copy(emb.at[pl.ds(0, BLK)], bufs[p], sins[p]).wait()
        pltpu.make_async_copy(lbl.at[pl.ds(0, RB)], las[p], sls[p]).wait()
        pltpu.make_async_copy(lbl.at[pl.ds(0, RB)], lbs[p], sls[p]).wait()

    def start_scat(p):
        da = pltpu.async_copy(bufs[p].at[pl.ds(0, RB)],
                              acc_sp.at[las[p]], sscs[p], add=True)
        db = pltpu.async_copy(bufs[p].at[pl.ds(RB, RB)],
                              acc_sp.at[lbs[p]], sscs[p], add=True)
        return da, db

    def compute(p):
        bf = bufs[p]
        for lb_ref in (las[p], lbs[p]):
            for t in range(8):
                l16 = lb_ref[pl.ds(t * 16, 16)]
                plsc.addupdate_scatter(cnt, [iota16, l16], ones16)

        def srow(r, a):
            for v in range(8):
                x0 = bf[2 * r, pl.ds(v * 16, 16)]
                x1 = bf[2 * r + 1, pl.ds(v * 16, 16)]
                a = a + x0 * x0 + x1 * x1
            return a
        blocksq = lax.fori_loop(0, BLK // 2, srow, jnp.zeros((16,), jnp.float32))
        sqbuf[pl.ds(0, 16)] = sqbuf[pl.ds(0, 16)] + blocksq

    for p in range(NRING):
        start_in(start + p, p)

    def tri_body(i, _):
        b0 = start + NRING * i
        for p in range(NRING):
            wait_in(p)
            d = start_scat(p)
            compute(p)
            d[0].wait()
            d[1].wait()

            @pl.when(i < TRIPS - 1)
            def _():
                start_in(b0 + p + NRING, p)
        return 0

    lax.fori_loop(0, TRIPS, tri_body, 0)

    pltpu.sync_copy(cnt, cnt_o.at[wid])
    pltpu.sync_copy(sqbuf, sq_o.at[wid])

    plsc.subcore_barrier()

    @pl.when(s == 0)
    def _():
        pltpu.sync_copy(acc_sp, part_o.at[c])


def _tc_body(emb_ref, lbl_ref, s_ref, cnt_ref, sq_ref):
    i = pl.program_id(0)
    x = emb_ref[...]
    lab = lbl_ref[0, 0, :]
    oh = (lab[:, None]
          == lax.broadcasted_iota(jnp.int32, (TCB, K), 1)).astype(jnp.bfloat16)
    hi = x.astype(jnp.bfloat16)
    lo = (x - hi.astype(jnp.float32)).astype(jnp.bfloat16)
    dn = (((0,), (0,)), ((), ()))
    sp = (lax.dot_general(oh, hi, dn, preferred_element_type=jnp.float32)
          + lax.dot_general(oh, lo, dn, preferred_element_type=jnp.float32))
    cp = jnp.sum(oh.astype(jnp.float32), axis=0)
    sqp = jnp.sum(x * x)

    @pl.when(i == 0)
    def _():
        s_ref[...] = jnp.zeros_like(s_ref)
        cnt_ref[...] = jnp.zeros_like(cnt_ref)
        sq_ref[...] = jnp.zeros_like(sq_ref)

    s_ref[...] += sp
    cnt_ref[...] += cp[None, :]
    sq_ref[...] += jnp.broadcast_to(sqp, (1, 1))


def _epi_body(part_ref, cnt_ref, sq_ref, stc_ref, ctc_ref, sqtc_ref, out_ref):
    S = part_ref[0] + part_ref[1] + stc_ref[...]
    rowsq = jnp.sum(S * S, axis=1)
    tot = jnp.sum(S, axis=0)
    tot2 = jnp.sum(tot * tot)
    counts = (jnp.sum(cnt_ref[...], axis=(0, 1)).astype(jnp.float32)
              + ctc_ref[0, :])
    present = counts > jnp.float32(0.5)
    k = jnp.sum(present.astype(jnp.int32))
    safe = jnp.where(present, counts, jnp.float32(1.0))
    T = jnp.sum(rowsq / safe)
    sumsq = jnp.sum(sq_ref[...]) + sqtc_ref[0, 0]
    n = jnp.float32(N)
    bcss = T - tot2 / n
    wcss = sumsq - T
    kf = k.astype(jnp.float32)
    ch = bcss * (n - kf) / ((kf - 1.0) * wcss + jnp.float32(1e-10))
    val = jnp.where((k < 2) | (k == N), jnp.float32(0.0), -ch)
    out_ref[...] = jnp.broadcast_to(val, (1, 1))


def kernel(embeddings, labels):
    labels = labels.reshape(-1)
    mesh = plsc.VectorSubcoreMesh(core_axis_name="c", subcore_axis_name="s")
    part, cnt, sq = pl.kernel(
        _sc_body,
        out_type=(
            jax.ShapeDtypeStruct((2, K, D), jnp.float32),
            jax.ShapeDtypeStruct((NW, 16, K), jnp.int32),
            jax.ShapeDtypeStruct((NW, 16), jnp.float32),
        ),
        mesh=mesh,
        compiler_params=pltpu.CompilerParams(needs_layout_passes=False),
        cost_estimate=pl.CostEstimate(
            flops=4 * SC_ROWS * D,
            transcendentals=0,
            bytes_accessed=SC_ROWS * D * 4 + SC_ROWS * 4,
        ),
        scratch_types=[
            pltpu.VMEM_SHARED((K, D), jnp.float32),
            pltpu.VMEM((BLK, D), jnp.float32),
            pltpu.VMEM((BLK, D), jnp.float32),
            pltpu.VMEM((BLK, D), jnp.float32),
            pltpu.VMEM((RB,), jnp.int32),
            pltpu.VMEM((RB,), jnp.int32),
            pltpu.VMEM((RB,), jnp.int32),
            pltpu.VMEM((RB,), jnp.int32),
            pltpu.VMEM((RB,), jnp.int32),
            pltpu.VMEM((RB,), jnp.int32),
            pltpu.VMEM((16, K), jnp.int32),
            pltpu.VMEM((16,), jnp.float32),
            pltpu.SemaphoreType.DMA,
            pltpu.SemaphoreType.DMA,
            pltpu.SemaphoreType.DMA,
            pltpu.SemaphoreType.DMA,
            pltpu.SemaphoreType.DMA,
            pltpu.SemaphoreType.DMA,
            pltpu.SemaphoreType.DMA,
            pltpu.SemaphoreType.DMA,
            pltpu.SemaphoreType.DMA,
        ],
    )(embeddings, labels)

    lbl3 = labels.reshape(N // TCB, 1, TCB)
    stc, ctc, sqtc = pl.pallas_call(
        _tc_body,
        grid=(TC_STEPS,),
        in_specs=[
            pl.BlockSpec((TCB, D), lambda i: (TC_START + i, 0)),
            pl.BlockSpec((1, 1, TCB), lambda i: (TC_START + i, 0, 0)),
        ],
        out_specs=[
            pl.BlockSpec((K, D), lambda i: (0, 0)),
            pl.BlockSpec((1, K), lambda i: (0, 0)),
            pl.BlockSpec((1, 1), lambda i: (0, 0)),
        ],
        out_shape=(
            jax.ShapeDtypeStruct((K, D), jnp.float32),
            jax.ShapeDtypeStruct((1, K), jnp.float32),
            jax.ShapeDtypeStruct((1, 1), jnp.float32),
        ),
    )(embeddings, lbl3)

    res = pl.pallas_call(
        _epi_body,
        out_shape=jax.ShapeDtypeStruct((1, 1), jnp.float32),
    )(part, cnt, sq, stc, ctc, sqtc)
    return jnp.reshape(res, ())


# speedup vs baseline: 1.2205x; 1.2205x over previous
import jax
import jax.numpy as jnp
from jax import lax
from jax.experimental import pallas as pl
from jax.experimental.pallas import tpu as pltpu
from jax.experimental.pallas import tpu_sc as plsc

N = 320000
D = 128
K = 1024
RB = 128
BLK = 256
NBLK = N // BLK
NW = 32
BASE_BLKS = NBLK // NW
EXTRA = NBLK - BASE_BLKS * NW
NRING = 3
TRIPS = BASE_BLKS // NRING


def _sc_body(emb, lbl, part_o, cnt_o, sq_o, acc_sp,
             buf0, buf1, buf2, la0, lb0, la1, lb1, la2, lb2, cnt, cntred,
             sqbuf, sin0, sin1, sin2, sl0, sl1, sl2, ssc0, ssc1, ssc2):
    c = lax.axis_index("c")
    s = lax.axis_index("s")
    wid = s * 2 + c

    zf16 = jnp.zeros((16,), jnp.float32)
    zi16 = jnp.zeros((16,), jnp.int32)
    iota16 = lax.broadcasted_iota(jnp.int32, (16,), 0)
    ones16 = jnp.ones((16,), jnp.int32)
    bufs = (buf0, buf1, buf2)
    las = (la0, la1, la2)
    lbs = (lb0, lb1, lb2)
    sins = (sin0, sin1, sin2)
    sls = (sl0, sl1, sl2)
    sscs = (ssc0, ssc1, ssc2)

    def zero_cnt(i, _):
        for j in range(16):
            cnt[j, pl.ds(i * 16, 16)] = zi16
        return 0
    lax.fori_loop(0, K // 16, zero_cnt, 0)
    sqbuf[pl.ds(0, 16)] = zf16

    def zero_buf(i, _):
        for v in range(8):
            buf0[i, pl.ds(v * 16, 16)] = zf16
        return 0
    lax.fori_loop(0, RB, zero_buf, 0)

    @pl.when(s == 0)
    def _():
        for i in range(K // RB):
            pltpu.sync_copy(buf0.at[pl.ds(0, RB)], acc_sp.at[pl.ds(i * RB, RB)])

    plsc.subcore_barrier()

    start = wid * BASE_BLKS

    def start_in(blkidx, p):
        row0 = blkidx * BLK
        pltpu.async_copy(emb.at[pl.ds(row0, BLK)], bufs[p], sins[p])
        pltpu.async_copy(lbl.at[pl.ds(row0, RB)], las[p], sls[p])
        pltpu.async_copy(lbl.at[pl.ds(row0 + RB, RB)], lbs[p], sls[p])

    def wait_in(p):
        pltpu.make_async_copy(emb.at[pl.ds(0, BLK)], bufs[p], sins[p]).wait()
        pltpu.make_async_copy(lbl.at[pl.ds(0, RB)], las[p], sls[p]).wait()
        pltpu.make_async_copy(lbl.at[pl.ds(0, RB)], lbs[p], sls[p]).wait()

    def start_scat(p):
        da = pltpu.async_copy(bufs[p].at[pl.ds(0, RB)],
                              acc_sp.at[las[p]], sscs[p], add=True)
        db = pltpu.async_copy(bufs[p].at[pl.ds(RB, RB)],
                              acc_sp.at[lbs[p]], sscs[p], add=True)
        return da, db

    def compute(p):
        bf = bufs[p]
        for lb_ref in (las[p], lbs[p]):
            for t in range(8):
                l16 = lb_ref[pl.ds(t * 16, 16)]
                plsc.addupdate_scatter(cnt, [iota16, l16], ones16)

        def srow(r, a):
            for v in range(8):
                x0 = bf[2 * r, pl.ds(v * 16, 16)]
                x1 = bf[2 * r + 1, pl.ds(v * 16, 16)]
                a = a + x0 * x0 + x1 * x1
            return a
        blocksq = lax.fori_loop(0, BLK // 2, srow, jnp.zeros((16,), jnp.float32))
        sqbuf[pl.ds(0, 16)] = sqbuf[pl.ds(0, 16)] + blocksq

    for p in range(NRING):
        start_in(start + p, p)

    def tri_body(i, _):
        b0 = start + NRING * i
        for p in range(NRING):
            wait_in(p)
            d = start_scat(p)
            compute(p)
            d[0].wait()
            d[1].wait()

            @pl.when(i < TRIPS - 1)
            def _():
                start_in(b0 + p + NRING, p)
        return 0

    lax.fori_loop(0, TRIPS, tri_body, 0)

    @pl.when(wid < EXTRA)
    def _():
        start_in(NW * BASE_BLKS + wid, 0)
        wait_in(0)
        pltpu.sync_copy(buf0.at[pl.ds(0, RB)], acc_sp.at[la0], add=True)
        pltpu.sync_copy(buf0.at[pl.ds(RB, RB)], acc_sp.at[lb0], add=True)
        compute(0)

    def red_cnt(g, _):
        a = cnt[0, pl.ds(g * 16, 16)]
        for j in range(1, 16):
            a = a + cnt[j, pl.ds(g * 16, 16)]
        cntred[pl.ds(g * 16, 16)] = a
        return 0
    lax.fori_loop(0, K // 16, red_cnt, 0)

    pltpu.sync_copy(cntred, cnt_o.at[wid])
    pltpu.sync_copy(sqbuf, sq_o.at[wid])

    plsc.subcore_barrier()

    @pl.when(s == 0)
    def _():
        pltpu.sync_copy(acc_sp, part_o.at[c])


def _epi_body(part_ref, cnt_ref, sq_ref, out_ref):
    S = part_ref[0] + part_ref[1]
    rowsq = jnp.sum(S * S, axis=1)
    tot = jnp.sum(S, axis=0)
    tot2 = jnp.sum(tot * tot)
    counts = jnp.sum(cnt_ref[...], axis=0)
    countsf = counts.astype(jnp.float32)
    present = counts > 0
    k = jnp.sum(present.astype(jnp.int32))
    safe = jnp.where(present, countsf, jnp.float32(1.0))
    T = jnp.sum(rowsq / safe)
    sumsq = jnp.sum(sq_ref[...])
    n = jnp.float32(N)
    bcss = T - tot2 / n
    wcss = sumsq - T
    kf = k.astype(jnp.float32)
    ch = bcss * (n - kf) / ((kf - 1.0) * wcss + jnp.float32(1e-10))
    val = jnp.where((k < 2) | (k == N), jnp.float32(0.0), -ch)
    out_ref[...] = jnp.broadcast_to(val, (1, 1))


def kernel(embeddings, labels):
    labels = labels.reshape(-1)
    mesh = plsc.VectorSubcoreMesh(core_axis_name="c", subcore_axis_name="s")
    part, cnt, sq = pl.kernel(
        _sc_body,
        out_type=(
            jax.ShapeDtypeStruct((2, K, D), jnp.float32),
            jax.ShapeDtypeStruct((NW, K), jnp.int32),
            jax.ShapeDtypeStruct((NW, 16), jnp.float32),
        ),
        mesh=mesh,
        compiler_params=pltpu.CompilerParams(needs_layout_passes=False),
        scratch_types=[
            pltpu.VMEM_SHARED((K, D), jnp.float32),
            pltpu.VMEM((BLK, D), jnp.float32),
            pltpu.VMEM((BLK, D), jnp.float32),
            pltpu.VMEM((BLK, D), jnp.float32),
            pltpu.VMEM((RB,), jnp.int32),
            pltpu.VMEM((RB,), jnp.int32),
            pltpu.VMEM((RB,), jnp.int32),
            pltpu.VMEM((RB,), jnp.int32),
            pltpu.VMEM((RB,), jnp.int32),
            pltpu.VMEM((RB,), jnp.int32),
            pltpu.VMEM((16, K), jnp.int32),
            pltpu.VMEM((K,), jnp.int32),
            pltpu.VMEM((16,), jnp.float32),
            pltpu.SemaphoreType.DMA,
            pltpu.SemaphoreType.DMA,
            pltpu.SemaphoreType.DMA,
            pltpu.SemaphoreType.DMA,
            pltpu.SemaphoreType.DMA,
            pltpu.SemaphoreType.DMA,
            pltpu.SemaphoreType.DMA,
            pltpu.SemaphoreType.DMA,
            pltpu.SemaphoreType.DMA,
        ],
    )(embeddings, labels)
    res = pl.pallas_call(
        _epi_body,
        out_shape=jax.ShapeDtypeStruct((1, 1), jnp.float32),
    )(part, cnt, sq)
    return jnp.reshape(res, ())
